# baseline (speedup 1.0000x reference)
import jax
import jax.numpy as jnp
import numpy as np
from jax.experimental import pallas as pl

N = 50000
B = 50
NPER = N // B
E = 1600000
K = int(np.ceil(0.2 * NPER))
NPOOL = B * K
HEADS = 3
CH = 20


def _gcn_conv(x, src, dst, W, b, n):
    loop = jnp.arange(n, dtype=src.dtype)
    s = jnp.concatenate([src, loop])
    d = jnp.concatenate([dst, loop])
    deg = jnp.zeros(n, x.dtype).at[d].add(1.0)
    dis = jnp.where(deg > 0, jax.lax.rsqrt(jnp.maximum(deg, 1e-12)), 0.0)
    norm = dis[s] * dis[d]
    xw = x @ W
    out = jnp.zeros((n, W.shape[1]), x.dtype).at[d].add(xw[s] * norm[:, None])
    return out + b


def _id_kernel(x_ref, o_ref):
    o_ref[...] = x_ref[...]


def kernel(x, edge_index, batch, W1, b1, Wp, bp, Wg, att_src, att_dst, bg, Wn1, bn1, Wn2, bn2):
    src = edge_index[0]
    dst = edge_index[1]
    h = _gcn_conv(x, src, dst, W1, b1, N)
    score = _gcn_conv(h, src, dst, Wp, bp, N)[:, 0]
    s2 = score.reshape(B, NPER)
    _, idx = jax.lax.top_k(s2, K)
    perm = (idx + (jnp.arange(B) * NPER)[:, None]).reshape(-1)
    xp = h[perm] * jnp.tanh(score[perm])[:, None]
    selected = jnp.zeros(N, bool).at[perm].set(True)
    node_map = jnp.zeros(N, jnp.int32).at[perm].set(jnp.arange(NPOOL, dtype=jnp.int32))
    em = selected[src] & selected[dst]
    ns = jnp.where(em, node_map[src], 0)
    nd = jnp.where(em, node_map[dst], 0)
    xl = (xp @ Wg).reshape(NPOOL, HEADS, CH)
    a_s = (xl * att_src).sum(-1)
    a_d = (xl * att_dst).sum(-1)
    loop = jnp.arange(NPOOL, dtype=jnp.int32)
    s_all = jnp.concatenate([ns, loop])
    d_all = jnp.concatenate([nd, loop])
    m_all = jnp.concatenate([em, jnp.ones(NPOOL, bool)])
    logit = jax.nn.leaky_relu(a_s[s_all] + a_d[d_all], 0.2)
    logit = jnp.where(m_all[:, None], logit, -1e9)
    mx = jax.ops.segment_max(logit, d_all, num_segments=NPOOL)
    ex = jnp.exp(logit - mx[d_all])
    ex = jnp.where(m_all[:, None], ex, 0.0)
    den = jax.ops.segment_sum(ex, d_all, num_segments=NPOOL)
    alpha = ex / (den[d_all] + 1e-16)
    msg = xl[s_all] * alpha[:, :, None]
    og = jax.ops.segment_sum(msg, d_all, num_segments=NPOOL).reshape(NPOOL, HEADS * CH) + bg
    batch_p = jnp.repeat(jnp.arange(B), K)
    g = jax.ops.segment_sum(og, batch_p, num_segments=B)
    h1 = jax.nn.relu(g @ Wn1 + bn1)
    out = jax.nn.log_softmax(h1 @ Wn2 + bn2, axis=1)
    return pl.pallas_call(
        _id_kernel, out_shape=jax.ShapeDtypeStruct(out.shape, out.dtype)
    )(out)


# speedup vs baseline: 1.0552x; 1.0552x over previous
import functools

import jax
import jax.numpy as jnp
import numpy as np
from jax import lax
from jax.experimental import pallas as pl
from jax.experimental.pallas import tpu as pltpu
from jax.experimental.pallas import tpu_sc as plsc

N = 50000
B = 50
NPER = N // B
E = 1600000
K = int(np.ceil(0.2 * NPER))
NPOOL = B * K
HEADS = 3
CH = 20

NC = 2
NS = 16
NW = NC * NS
L = 16

NP = 51200
ECHUNK = 2048
EPT = 25 * ECHUNK
EP = NW * EPT
CE = 512
NPP = 10240
BLKN = 2048


def _mesh():
    return plsc.VectorSubcoreMesh(core_axis_name="c", subcore_axis_name="s",
                                  num_cores=NC, num_subcores=NS)


_SC_PARAMS = pltpu.CompilerParams(needs_layout_passes=False)
_SC_PARAMS_NT = pltpu.CompilerParams(needs_layout_passes=False,
                                     use_tc_tiling_on_sc=False)


def _zero_vmem(ref, nwords):
    z16 = jnp.zeros((L,), jnp.float32)

    def zbody(i, _):
        ref[pl.ds(i * L, L)] = z16
        return 0

    lax.fori_loop(0, nwords // L, zbody, 0)


def _deg_count(dst_p):
    @functools.partial(
        pl.kernel,
        out_type=jax.ShapeDtypeStruct((NW, NP), jnp.float32),
        mesh=_mesh(),
        compiler_params=_SC_PARAMS,
        scratch_types=[
            pltpu.VMEM((ECHUNK,), jnp.int32),
            pltpu.VMEM((NP,), jnp.float32),
            pltpu.SemaphoreType.DMA,
        ],
    )
    def deg_kernel(dst_hbm, out_hbm, dstbuf, acc, sem):
        cid = lax.axis_index("c")
        sid = lax.axis_index("s")
        wid = sid * NC + cid
        ones16 = jnp.ones((L,), jnp.float32)
        _zero_vmem(acc, NP)
        base = wid * EPT

        def chunk_body(k, _):
            pltpu.sync_copy(dst_hbm.at[pl.ds(base + k * ECHUNK, ECHUNK)], dstbuf)

            def body(i, _):
                idx = dstbuf[pl.ds(i * L, L)]
                plsc.addupdate_scatter(acc, [idx], ones16)
                return 0

            lax.fori_loop(0, ECHUNK // L, body, 0)
            return 0

        lax.fori_loop(0, EPT // ECHUNK, chunk_body, 0)
        pltpu.sync_copy(acc, out_hbm.at[wid])

    return deg_kernel(dst_p)


def _dis(degp):
    NL = NP // 128

    def body(degp_ref, dis_ref):
        deg = jnp.sum(degp_ref[...], axis=0) + 1.0
        dis_ref[...] = lax.rsqrt(deg)

    return pl.pallas_call(
        body,
        grid=(25,),
        in_specs=[pl.BlockSpec((NW, NL // 25, 128), lambda i: (0, i, 0))],
        out_specs=pl.BlockSpec((NL // 25, 128), lambda i: (i, 0)),
        out_shape=jax.ShapeDtypeStruct((NL, 128), jnp.float32),
    )(degp.reshape(NW, NL, 128))


def _prep(disR, x5T):
    def body(dis_ref, x_ref, y_ref):
        y_ref[...] = x_ref[...] * dis_ref[...]

    grid = NP // BLKN
    return pl.pallas_call(
        body,
        grid=(grid,),
        in_specs=[
            pl.BlockSpec((1, BLKN), lambda i: (0, i)),
            pl.BlockSpec((5, BLKN), lambda i: (0, i)),
        ],
        out_specs=pl.BlockSpec((5, BLKN), lambda i: (0, i)),
        out_shape=jax.ShapeDtypeStruct((5, NP), jnp.float32),
    )(disR, x5T)


def _row_agg(src_p, dst_p, yT):
    CHN = 5

    @functools.partial(
        pl.kernel,
        out_type=jax.ShapeDtypeStruct((CHN * NW, NP), jnp.float32),
        mesh=_mesh(),
        compiler_params=_SC_PARAMS,
        scratch_types=[
            pltpu.VMEM((ECHUNK,), jnp.int32),
            pltpu.VMEM((ECHUNK,), jnp.int32),
            pltpu.VMEM((NP,), jnp.float32),
            pltpu.VMEM((NP,), jnp.float32),
            pltpu.SemaphoreType.DMA,
        ],
    )
    def row_kernel(src_hbm, dst_hbm, y_hbm, out_hbm, srcbuf, dstbuf, ztab,
                   acc, sem):
        cid = lax.axis_index("c")
        sid = lax.axis_index("s")
        wid = sid * NC + cid
        base = wid * EPT
        for ch in range(CHN):
            _zero_vmem(acc, NP)
            pltpu.sync_copy(y_hbm.at[pl.ds(ch * NP, NP)], ztab)

            def chunk_body(k, _):
                pltpu.sync_copy(src_hbm.at[pl.ds(base + k * ECHUNK, ECHUNK)],
                                srcbuf)
                pltpu.sync_copy(dst_hbm.at[pl.ds(base + k * ECHUNK, ECHUNK)],
                                dstbuf)

                def body(i, _):
                    s16 = srcbuf[pl.ds(i * L, L)]
                    d16 = dstbuf[pl.ds(i * L, L)]
                    zi = plsc.load_gather(ztab, [s16])
                    plsc.addupdate_scatter(acc, [d16], zi)
                    return 0

                lax.fori_loop(0, ECHUNK // L, body, 0)
                return 0

            lax.fori_loop(0, EPT // ECHUNK, chunk_body, 0)
            pltpu.sync_copy(acc, out_hbm.at[ch * NW + wid])

    return row_kernel(src_p, dst_p, yT.reshape(CHN * NP)).reshape(
        CHN, NW, NP)


def _hz(aggT, yT, disR, dis2d, W1p5, b1r, Wp16):
    def body(aggT_ref, y_ref, disr_ref, dis_ref, w1_ref, b1_ref, wp_ref,
             h_ref, z_ref):
        aggsum = jnp.sum(aggT_ref[...], axis=1)
        t5 = (aggsum + y_ref[...]) * disr_ref[...]
        h = lax.dot_general(t5, w1_ref[...], (((0,), (0,)), ((), ())),
                            preferred_element_type=jnp.float32)
        h = h + b1_ref[...]
        h_ref[...] = h
        z_ref[...] = jnp.dot(h, wp_ref[...],
                             preferred_element_type=jnp.float32) * dis_ref[...]

    grid = NP // BLKN
    return pl.pallas_call(
        body,
        grid=(grid,),
        in_specs=[
            pl.BlockSpec((5, NW, BLKN), lambda i: (0, 0, i)),
            pl.BlockSpec((5, BLKN), lambda i: (0, i)),
            pl.BlockSpec((1, BLKN), lambda i: (0, i)),
            pl.BlockSpec((BLKN, 1), lambda i: (i, 0)),
            pl.BlockSpec((5, 16), lambda i: (0, 0)),
            pl.BlockSpec((1, 16), lambda i: (0, 0)),
            pl.BlockSpec((16, 1), lambda i: (0, 0)),
        ],
        out_specs=[
            pl.BlockSpec((BLKN, 16), lambda i: (i, 0)),
            pl.BlockSpec((BLKN, 1), lambda i: (i, 0)),
        ],
        out_shape=[
            jax.ShapeDtypeStruct((NP, 16), jnp.float32),
            jax.ShapeDtypeStruct((NP, 1), jnp.float32),
        ],
    )(aggT, yT, disR, dis2d, W1p5, b1r, Wp16)


def _score_agg(src_p, dst_p, z1d):
    @functools.partial(
        pl.kernel,
        out_type=jax.ShapeDtypeStruct((NW, NP), jnp.float32),
        mesh=_mesh(),
        compiler_params=_SC_PARAMS,
        scratch_types=[
            pltpu.VMEM((ECHUNK,), jnp.int32),
            pltpu.VMEM((ECHUNK,), jnp.int32),
            pltpu.VMEM((NP,), jnp.float32),
            pltpu.VMEM((NP,), jnp.float32),
            pltpu.SemaphoreType.DMA,
        ],
    )
    def sagg_kernel(src_hbm, dst_hbm, z_hbm, out_hbm, srcbuf, dstbuf, ztab,
                    acc, sem):
        cid = lax.axis_index("c")
        sid = lax.axis_index("s")
        wid = sid * NC + cid
        _zero_vmem(acc, NP)
        pltpu.sync_copy(z_hbm, ztab)
        base = wid * EPT

        def chunk_body(k, _):
            pltpu.sync_copy(src_hbm.at[pl.ds(base + k * ECHUNK, ECHUNK)], srcbuf)
            pltpu.sync_copy(dst_hbm.at[pl.ds(base + k * ECHUNK, ECHUNK)], dstbuf)

            def body(i, _):
                s16 = srcbuf[pl.ds(i * L, L)]
                d16 = dstbuf[pl.ds(i * L, L)]
                zi = plsc.load_gather(ztab, [s16])
                plsc.addupdate_scatter(acc, [d16], zi)
                return 0

            lax.fori_loop(0, ECHUNK // L, body, 0)
            return 0

        lax.fori_loop(0, EPT // ECHUNK, chunk_body, 0)
        pltpu.sync_copy(acc, out_hbm.at[wid])

    return sagg_kernel(src_p, dst_p, z1d)


def _topk(saggp3, z3, dis3, bp):
    KF = float(K)

    def body(sagg_ref, z_ref, dis_ref, bp_ref, nm_ref, tsc_ref):
        sagg = jnp.sum(sagg_ref[...], axis=0)
        score = dis_ref[...] * (sagg + z_ref[...]) + bp_ref[0, 0]
        tsc_ref[...] = jnp.tanh(score)
        u = lax.bitcast_convert_type(score, jnp.uint32)
        top = jnp.uint32(0x80000000)
        key = jnp.where(u >= top, ~u, u | top)

        def bit_body(i, T):
            b = 31 - i
            cand = T | (jnp.uint32(1) << b)
            cnt = jnp.sum(jnp.where(key >= cand, 1.0, 0.0), axis=1,
                          keepdims=True)
            return jnp.where(cnt >= KF, cand, T)

        T = lax.fori_loop(0, 32, bit_body, jnp.zeros((B, 1), jnp.uint32))
        gt = key > T
        eq = key == T
        gtc = jnp.sum(jnp.where(gt, 1.0, 0.0), axis=1, keepdims=True)
        need = KF - gtc
        r = lax.broadcasted_iota(jnp.int32, (NPER, NPER), 0)
        c = lax.broadcasted_iota(jnp.int32, (NPER, NPER), 1)
        triu = jnp.where(r <= c, 1.0, 0.0).astype(jnp.float32)
        eqf = jnp.where(eq, 1.0, 0.0)
        cum = jnp.dot(eqf, triu, preferred_element_type=jnp.float32)
        sel = gt | (eq & (cum <= need))
        self_ = jnp.where(sel, 1.0, 0.0)
        selcum = jnp.dot(self_, triu, preferred_element_type=jnp.float32)
        rowbase = lax.broadcasted_iota(jnp.int32, (B, NPER), 0) * K
        nm_ref[...] = jnp.where(
            sel, rowbase + selcum.astype(jnp.int32) - 1, NPOOL)

    return pl.pallas_call(
        body,
        in_specs=[
            pl.BlockSpec((NW, B, NPER), lambda: (0, 0, 0)),
            pl.BlockSpec((B, NPER), lambda: (0, 0)),
            pl.BlockSpec((B, NPER), lambda: (0, 0)),
            pl.BlockSpec((1, 1), lambda: (0, 0)),
        ],
        out_specs=[
            pl.BlockSpec((B, NPER), lambda: (0, 0)),
            pl.BlockSpec((B, NPER), lambda: (0, 0)),
        ],
        out_shape=[
            jax.ShapeDtypeStruct((B, NPER), jnp.int32),
            jax.ShapeDtypeStruct((B, NPER), jnp.float32),
        ],
    )(saggp3, z3, dis3, bp.reshape(1, 1))


def _ht(h16, t2d):
    def body(h_ref, t_ref, o_ref):
        o_ref[...] = h_ref[...] * t_ref[...]

    grid = NP // BLKN
    return pl.pallas_call(
        body,
        grid=(grid,),
        in_specs=[
            pl.BlockSpec((BLKN, 16), lambda i: (i, 0)),
            pl.BlockSpec((BLKN, 1), lambda i: (i, 0)),
        ],
        out_specs=pl.BlockSpec((BLKN, 16), lambda i: (i, 0)),
        out_shape=jax.ShapeDtypeStruct((NP, 16), jnp.float32),
    )(h16, t2d)


def kernel(x, edge_index, batch, W1, b1, Wp, bp, Wg, att_src, att_dst, bg, Wn1, bn1, Wn2, bn2):
    src = edge_index[0]
    dst = edge_index[1]
    pad = jnp.full((EP - E,), N, jnp.int32)
    src_p = jnp.concatenate([src, pad])
    dst_p = jnp.concatenate([dst, pad])
    x5T = jnp.zeros((5, NP), jnp.float32).at[:, :N].set(x.T)
    W1p5 = jnp.zeros((5, 16), jnp.float32).at[:, :10].set(W1)
    b1r = jnp.zeros((1, 16), jnp.float32).at[0, :10].set(b1)
    Wp16 = jnp.zeros((16, 1), jnp.float32).at[:10].set(Wp)

    degp = _deg_count(dst_p)
    dis2d = _dis(degp).reshape(NP, 1)
    disR = dis2d.reshape(1, NP)
    yT = _prep(disR, x5T)
    aggT = _row_agg(src_p, dst_p, yT)
    h16, z2d = _hz(aggT, yT, disR, dis2d, W1p5, b1r, Wp16)
    saggp = _score_agg(src_p, dst_p, z2d.reshape(NP))

    saggp3 = saggp[:, :N].reshape(NW, B, NPER)
    z3 = z2d[:N].reshape(B, NPER)
    dis3 = dis2d[:N].reshape(B, NPER)
    nm, tsc = _topk(saggp3, z3, dis3, bp)

    t2d = jnp.zeros((NP, 1), jnp.float32).at[:N, 0].set(tsc.reshape(N))
    ht16 = _ht(h16, t2d)

    nm_flat = nm.reshape(N)
    selected = nm_flat < NPOOL
    perm = jnp.zeros((NPOOL,), jnp.int32).at[
        jnp.where(selected, nm_flat, NPOOL)].set(
            jnp.arange(N, dtype=jnp.int32), mode="drop")
    xp = ht16[perm, :10]
    em = selected[src] & selected[dst]
    ns = jnp.where(em, nm_flat[src], 0)
    nd = jnp.where(em, nm_flat[dst], 0)
    xl = (xp @ Wg).reshape(NPOOL, HEADS, CH)
    a_s = (xl * att_src).sum(-1)
    a_d = (xl * att_dst).sum(-1)
    loop = jnp.arange(NPOOL, dtype=jnp.int32)
    s_all = jnp.concatenate([ns, loop])
    d_all = jnp.concatenate([nd, loop])
    m_all = jnp.concatenate([em, jnp.ones(NPOOL, bool)])
    logit = jax.nn.leaky_relu(a_s[s_all] + a_d[d_all], 0.2)
    logit = jnp.where(m_all[:, None], logit, -1e9)
    mx = jax.ops.segment_max(logit, d_all, num_segments=NPOOL)
    ex = jnp.exp(logit - mx[d_all])
    ex = jnp.where(m_all[:, None], ex, 0.0)
    den = jax.ops.segment_sum(ex, d_all, num_segments=NPOOL)
    alpha = ex / (den[d_all] + 1e-16)
    msg = xl[s_all] * alpha[:, :, None]
    og = jax.ops.segment_sum(msg, d_all, num_segments=NPOOL).reshape(NPOOL, HEADS * CH) + bg
    batch_p = jnp.repeat(jnp.arange(B), K)
    g = jax.ops.segment_sum(og, batch_p, num_segments=B)
    h1 = jax.nn.relu(g @ Wn1 + bn1)
    return jax.nn.log_softmax(h1 @ Wn2 + bn2, axis=1)


# speedup vs baseline: 3.0322x; 2.8736x over previous
import functools

import jax
import jax.numpy as jnp
import numpy as np
from jax import lax
from jax.experimental import pallas as pl
from jax.experimental.pallas import tpu as pltpu
from jax.experimental.pallas import tpu_sc as plsc

N = 50000
B = 50
NPER = N // B
E = 1600000
K = int(np.ceil(0.2 * NPER))
NPOOL = B * K
HEADS = 3
CH = 20

NC = 2
NS = 16
NW = NC * NS
L = 16

NP = 51200
ECHUNK = 2048
EPT = 25 * ECHUNK
EP = NW * EPT
CE = 512
NPP = 10240
BLKN = 2048


def _mesh():
    return plsc.VectorSubcoreMesh(core_axis_name="c", subcore_axis_name="s",
                                  num_cores=NC, num_subcores=NS)


_SC_PARAMS = pltpu.CompilerParams(needs_layout_passes=False)
_SC_PARAMS_NT = pltpu.CompilerParams(needs_layout_passes=False,
                                     use_tc_tiling_on_sc=False)


def _zero_vmem(ref, nwords):
    z16 = jnp.zeros((L,), jnp.float32)

    def zbody(i, _):
        ref[pl.ds(i * L, L)] = z16
        return 0

    lax.fori_loop(0, nwords // L, zbody, 0)


def _deg_count(dst_p):
    @functools.partial(
        pl.kernel,
        out_type=jax.ShapeDtypeStruct((NW, NP), jnp.float32),
        mesh=_mesh(),
        compiler_params=_SC_PARAMS,
        scratch_types=[
            pltpu.VMEM((ECHUNK,), jnp.int32),
            pltpu.VMEM((NP,), jnp.float32),
            pltpu.SemaphoreType.DMA,
        ],
    )
    def deg_kernel(dst_hbm, out_hbm, dstbuf, acc, sem):
        cid = lax.axis_index("c")
        sid = lax.axis_index("s")
        wid = sid * NC + cid
        ones16 = jnp.ones((L,), jnp.float32)
        _zero_vmem(acc, NP)
        base = wid * EPT

        def chunk_body(k, _):
            pltpu.sync_copy(dst_hbm.at[pl.ds(base + k * ECHUNK, ECHUNK)], dstbuf)

            def body(i, _):
                idx = dstbuf[pl.ds(i * L, L)]
                plsc.addupdate_scatter(acc, [idx], ones16)
                return 0

            lax.fori_loop(0, ECHUNK // L, body, 0)
            return 0

        lax.fori_loop(0, EPT // ECHUNK, chunk_body, 0)
        pltpu.sync_copy(acc, out_hbm.at[wid])

    return deg_kernel(dst_p)


def _dis(degp):
    NL = NP // 128

    def body(degp_ref, dis_ref):
        deg = jnp.sum(degp_ref[...], axis=0) + 1.0
        dis_ref[...] = lax.rsqrt(deg)

    return pl.pallas_call(
        body,
        grid=(25,),
        in_specs=[pl.BlockSpec((NW, NL // 25, 128), lambda i: (0, i, 0))],
        out_specs=pl.BlockSpec((NL // 25, 128), lambda i: (i, 0)),
        out_shape=jax.ShapeDtypeStruct((NL, 128), jnp.float32),
    )(degp.reshape(NW, NL, 128))


def _prep(disR, x5T):
    def body(dis_ref, x_ref, y_ref):
        y_ref[...] = x_ref[...] * dis_ref[...]

    grid = NP // BLKN
    return pl.pallas_call(
        body,
        grid=(grid,),
        in_specs=[
            pl.BlockSpec((1, BLKN), lambda i: (0, i)),
            pl.BlockSpec((5, BLKN), lambda i: (0, i)),
        ],
        out_specs=pl.BlockSpec((5, BLKN), lambda i: (0, i)),
        out_shape=jax.ShapeDtypeStruct((5, NP), jnp.float32),
    )(disR, x5T)


def _row_agg(src_p, dst_p, yT):
    CHN = 5

    @functools.partial(
        pl.kernel,
        out_type=jax.ShapeDtypeStruct((CHN * NW, NP), jnp.float32),
        mesh=_mesh(),
        compiler_params=_SC_PARAMS,
        scratch_types=[
            pltpu.VMEM((ECHUNK,), jnp.int32),
            pltpu.VMEM((ECHUNK,), jnp.int32),
            pltpu.VMEM((NP,), jnp.float32),
            pltpu.VMEM((NP,), jnp.float32),
            pltpu.SemaphoreType.DMA,
        ],
    )
    def row_kernel(src_hbm, dst_hbm, y_hbm, out_hbm, srcbuf, dstbuf, ztab,
                   acc, sem):
        cid = lax.axis_index("c")
        sid = lax.axis_index("s")
        wid = sid * NC + cid
        base = wid * EPT
        for ch in range(CHN):
            _zero_vmem(acc, NP)
            pltpu.sync_copy(y_hbm.at[pl.ds(ch * NP, NP)], ztab)

            def chunk_body(k, _):
                pltpu.sync_copy(src_hbm.at[pl.ds(base + k * ECHUNK, ECHUNK)],
                                srcbuf)
                pltpu.sync_copy(dst_hbm.at[pl.ds(base + k * ECHUNK, ECHUNK)],
                                dstbuf)

                def body(i, _):
                    s16 = srcbuf[pl.ds(i * L, L)]
                    d16 = dstbuf[pl.ds(i * L, L)]
                    zi = plsc.load_gather(ztab, [s16])
                    plsc.addupdate_scatter(acc, [d16], zi)
                    return 0

                lax.fori_loop(0, ECHUNK // L, body, 0)
                return 0

            lax.fori_loop(0, EPT // ECHUNK, chunk_body, 0)
            pltpu.sync_copy(acc, out_hbm.at[ch * NW + wid])

    return row_kernel(src_p, dst_p, yT.reshape(CHN * NP)).reshape(
        CHN, NW, NP)


def _hz(aggT, yT, disR, dis2d, W1p5, b1r, Wp16):
    def body(aggT_ref, y_ref, disr_ref, dis_ref, w1_ref, b1_ref, wp_ref,
             h_ref, z_ref):
        aggsum = jnp.sum(aggT_ref[...], axis=1)
        t5 = (aggsum + y_ref[...]) * disr_ref[...]
        h = lax.dot_general(t5, w1_ref[...], (((0,), (0,)), ((), ())),
                            preferred_element_type=jnp.float32)
        h = h + b1_ref[...]
        h_ref[...] = h
        z_ref[...] = jnp.dot(h, wp_ref[...],
                             preferred_element_type=jnp.float32) * dis_ref[...]

    grid = NP // BLKN
    return pl.pallas_call(
        body,
        grid=(grid,),
        in_specs=[
            pl.BlockSpec((5, NW, BLKN), lambda i: (0, 0, i)),
            pl.BlockSpec((5, BLKN), lambda i: (0, i)),
            pl.BlockSpec((1, BLKN), lambda i: (0, i)),
            pl.BlockSpec((BLKN, 1), lambda i: (i, 0)),
            pl.BlockSpec((5, 16), lambda i: (0, 0)),
            pl.BlockSpec((1, 16), lambda i: (0, 0)),
            pl.BlockSpec((16, 1), lambda i: (0, 0)),
        ],
        out_specs=[
            pl.BlockSpec((BLKN, 16), lambda i: (i, 0)),
            pl.BlockSpec((BLKN, 1), lambda i: (i, 0)),
        ],
        out_shape=[
            jax.ShapeDtypeStruct((NP, 16), jnp.float32),
            jax.ShapeDtypeStruct((NP, 1), jnp.float32),
        ],
    )(aggT, yT, disR, dis2d, W1p5, b1r, Wp16)


def _score_agg(src_p, dst_p, z1d):
    @functools.partial(
        pl.kernel,
        out_type=jax.ShapeDtypeStruct((NW, NP), jnp.float32),
        mesh=_mesh(),
        compiler_params=_SC_PARAMS,
        scratch_types=[
            pltpu.VMEM((ECHUNK,), jnp.int32),
            pltpu.VMEM((ECHUNK,), jnp.int32),
            pltpu.VMEM((NP,), jnp.float32),
            pltpu.VMEM((NP,), jnp.float32),
            pltpu.SemaphoreType.DMA,
        ],
    )
    def sagg_kernel(src_hbm, dst_hbm, z_hbm, out_hbm, srcbuf, dstbuf, ztab,
                    acc, sem):
        cid = lax.axis_index("c")
        sid = lax.axis_index("s")
        wid = sid * NC + cid
        _zero_vmem(acc, NP)
        pltpu.sync_copy(z_hbm, ztab)
        base = wid * EPT

        def chunk_body(k, _):
            pltpu.sync_copy(src_hbm.at[pl.ds(base + k * ECHUNK, ECHUNK)], srcbuf)
            pltpu.sync_copy(dst_hbm.at[pl.ds(base + k * ECHUNK, ECHUNK)], dstbuf)

            def body(i, _):
                s16 = srcbuf[pl.ds(i * L, L)]
                d16 = dstbuf[pl.ds(i * L, L)]
                zi = plsc.load_gather(ztab, [s16])
                plsc.addupdate_scatter(acc, [d16], zi)
                return 0

            lax.fori_loop(0, ECHUNK // L, body, 0)
            return 0

        lax.fori_loop(0, EPT // ECHUNK, chunk_body, 0)
        pltpu.sync_copy(acc, out_hbm.at[wid])

    return sagg_kernel(src_p, dst_p, z1d)


def _topk(saggp3, z3, dis3, bp):
    KF = float(K)

    def body(sagg_ref, z_ref, dis_ref, bp_ref, nm_ref, tsc_ref):
        sagg = jnp.sum(sagg_ref[...], axis=0)
        score = dis_ref[...] * (sagg + z_ref[...]) + bp_ref[0, 0]
        tsc_ref[...] = jnp.tanh(score)
        u = lax.bitcast_convert_type(score, jnp.uint32)
        top = jnp.uint32(0x80000000)
        key = jnp.where(u >= top, ~u, u | top)

        def bit_body(i, T):
            b = 31 - i
            cand = T | (jnp.uint32(1) << b)
            cnt = jnp.sum(jnp.where(key >= cand, 1.0, 0.0), axis=1,
                          keepdims=True)
            return jnp.where(cnt >= KF, cand, T)

        T = lax.fori_loop(0, 32, bit_body, jnp.zeros((B, 1), jnp.uint32))
        gt = key > T
        eq = key == T
        gtc = jnp.sum(jnp.where(gt, 1.0, 0.0), axis=1, keepdims=True)
        need = KF - gtc
        r = lax.broadcasted_iota(jnp.int32, (NPER, NPER), 0)
        c = lax.broadcasted_iota(jnp.int32, (NPER, NPER), 1)
        triu = jnp.where(r <= c, 1.0, 0.0).astype(jnp.float32)
        eqf = jnp.where(eq, 1.0, 0.0)
        cum = jnp.dot(eqf, triu, preferred_element_type=jnp.float32)
        sel = gt | (eq & (cum <= need))
        self_ = jnp.where(sel, 1.0, 0.0)
        selcum = jnp.dot(self_, triu, preferred_element_type=jnp.float32)
        rowbase = lax.broadcasted_iota(jnp.int32, (B, NPER), 0) * K
        nm_ref[...] = jnp.where(
            sel, rowbase + selcum.astype(jnp.int32) - 1, NPOOL)

    return pl.pallas_call(
        body,
        in_specs=[
            pl.BlockSpec((NW, B, NPER), lambda: (0, 0, 0)),
            pl.BlockSpec((B, NPER), lambda: (0, 0)),
            pl.BlockSpec((B, NPER), lambda: (0, 0)),
            pl.BlockSpec((1, 1), lambda: (0, 0)),
        ],
        out_specs=[
            pl.BlockSpec((B, NPER), lambda: (0, 0)),
            pl.BlockSpec((B, NPER), lambda: (0, 0)),
        ],
        out_shape=[
            jax.ShapeDtypeStruct((B, NPER), jnp.int32),
            jax.ShapeDtypeStruct((B, NPER), jnp.float32),
        ],
    )(saggp3, z3, dis3, bp.reshape(1, 1))


def _ht(h16, t2d):
    def body(h_ref, t_ref, o_ref):
        o_ref[...] = h_ref[...] * t_ref[...]

    grid = NP // BLKN
    return pl.pallas_call(
        body,
        grid=(grid,),
        in_specs=[
            pl.BlockSpec((BLKN, 16), lambda i: (i, 0)),
            pl.BlockSpec((BLKN, 1), lambda i: (i, 0)),
        ],
        out_specs=pl.BlockSpec((BLKN, 16), lambda i: (i, 0)),
        out_shape=jax.ShapeDtypeStruct((NP, 16), jnp.float32),
    )(h16, t2d)


def _gat_msg(ns_p, nd_p, awT, xlT):
    CHG = 60

    @functools.partial(
        pl.kernel,
        out_type=jax.ShapeDtypeStruct((CHG * NW, NPP), jnp.float32),
        mesh=_mesh(),
        compiler_params=_SC_PARAMS,
        scratch_types=[
            pltpu.VMEM((ECHUNK,), jnp.int32),
            pltpu.VMEM((ECHUNK,), jnp.int32),
            pltpu.VMEM((ECHUNK,), jnp.float32),
            pltpu.VMEM((NPP,), jnp.float32),
            pltpu.VMEM((NPP,), jnp.float32),
            pltpu.SemaphoreType.DMA,
        ],
    )
    def msg_kernel(ns_hbm, nd_hbm, aw_hbm, xl_hbm, out_hbm, nsbuf, ndbuf,
                   awbuf, ztab, acc, sem):
        cid = lax.axis_index("c")
        sid = lax.axis_index("s")
        wid = sid * NC + cid
        base = wid * EPT
        for ch in range(CHG):
            hd = min(ch // CH, HEADS - 1)
            _zero_vmem(acc, NPP)
            pltpu.sync_copy(xl_hbm.at[pl.ds(ch * NPP, NPP)], ztab)

            def chunk_body(k, _):
                off = base + k * ECHUNK
                pltpu.sync_copy(ns_hbm.at[pl.ds(off, ECHUNK)], nsbuf)
                pltpu.sync_copy(nd_hbm.at[pl.ds(off, ECHUNK)], ndbuf)
                pltpu.sync_copy(aw_hbm.at[pl.ds(hd * EP + off, ECHUNK)], awbuf)

                def body(i, _):
                    s16 = nsbuf[pl.ds(i * L, L)]
                    d16 = ndbuf[pl.ds(i * L, L)]
                    w16 = awbuf[pl.ds(i * L, L)]
                    v = plsc.load_gather(ztab, [s16]) * w16
                    plsc.addupdate_scatter(acc, [d16], v)
                    return 0

                lax.fori_loop(0, ECHUNK // L, body, 0)
                return 0

            lax.fori_loop(0, EPT // ECHUNK, chunk_body, 0)
            pltpu.sync_copy(acc, out_hbm.at[ch * NW + wid])

    return msg_kernel(ns_p, nd_p, awT, xlT)


def _combine(ogp3, xlTp, w3Tp, bgc):
    CHG = 60
    BL = 1024

    def body(og_ref, xl_ref, w3_ref, bg_ref, o_ref):
        og = jnp.sum(og_ref[...], axis=1)
        rows = lax.broadcasted_iota(jnp.int32, (CHG, HEADS), 0) // CH
        cols = lax.broadcasted_iota(jnp.int32, (CHG, HEADS), 1)
        sel = jnp.where(rows == cols, 1.0, 0.0).astype(jnp.float32)
        wmat = jnp.dot(sel, w3_ref[...], preferred_element_type=jnp.float32)
        o_ref[...] = og + xl_ref[...] * wmat + bg_ref[...]

    return pl.pallas_call(
        body,
        grid=(NPP // BL,),
        in_specs=[
            pl.BlockSpec((CHG, NW, BL), lambda i: (0, 0, i)),
            pl.BlockSpec((CHG, BL), lambda i: (0, i)),
            pl.BlockSpec((HEADS, BL), lambda i: (0, i)),
            pl.BlockSpec((CHG, 1), lambda i: (0, 0)),
        ],
        out_specs=pl.BlockSpec((CHG, BL), lambda i: (0, i)),
        out_shape=jax.ShapeDtypeStruct((CHG, NPP), jnp.float32),
    )(ogp3, xlTp, w3Tp, bgc)


def _headout(ogt3, Wn1T, bn1c, Wn2T, bn2c):
    def body(og_ref, w1_ref, b1_ref, w2_ref, b2_ref, o_ref):
        g = jnp.sum(og_ref[...], axis=2)
        h1 = jnp.maximum(
            jnp.dot(w1_ref[...], g,
                    preferred_element_type=jnp.float32) + b1_ref[...], 0.0)
        lg = jnp.dot(w2_ref[...], h1,
                     preferred_element_type=jnp.float32) + b2_ref[...]
        rows = lax.broadcasted_iota(jnp.int32, (8, B), 0)
        zm = jnp.where(rows < 3, lg, -1e30)
        mx = jnp.max(zm, axis=0, keepdims=True)
        e = jnp.exp(zm - mx)
        den = jnp.sum(e, axis=0, keepdims=True)
        o_ref[...] = zm - mx - jnp.log(den)

    return pl.pallas_call(
        body,
        in_specs=[
            pl.BlockSpec((60, B, K), lambda: (0, 0, 0)),
            pl.BlockSpec((32, 60), lambda: (0, 0)),
            pl.BlockSpec((32, 1), lambda: (0, 0)),
            pl.BlockSpec((8, 32), lambda: (0, 0)),
            pl.BlockSpec((8, 1), lambda: (0, 0)),
        ],
        out_specs=pl.BlockSpec((8, B), lambda: (0, 0)),
        out_shape=jax.ShapeDtypeStruct((8, B), jnp.float32),
    )(ogt3, Wn1T, bn1c, Wn2T, bn2c)


def kernel(x, edge_index, batch, W1, b1, Wp, bp, Wg, att_src, att_dst, bg, Wn1, bn1, Wn2, bn2):
    src = edge_index[0]
    dst = edge_index[1]
    pad = jnp.full((EP - E,), N, jnp.int32)
    src_p = jnp.concatenate([src, pad])
    dst_p = jnp.concatenate([dst, pad])
    x5T = jnp.zeros((5, NP), jnp.float32).at[:, :N].set(x.T)
    W1p5 = jnp.zeros((5, 16), jnp.float32).at[:, :10].set(W1)
    b1r = jnp.zeros((1, 16), jnp.float32).at[0, :10].set(b1)
    Wp16 = jnp.zeros((16, 1), jnp.float32).at[:10].set(Wp)

    degp = _deg_count(dst_p)
    dis2d = _dis(degp).reshape(NP, 1)
    disR = dis2d.reshape(1, NP)
    yT = _prep(disR, x5T)
    aggT = _row_agg(src_p, dst_p, yT)
    h16, z2d = _hz(aggT, yT, disR, dis2d, W1p5, b1r, Wp16)
    saggp = _score_agg(src_p, dst_p, z2d.reshape(NP))

    saggp3 = saggp[:, :N].reshape(NW, B, NPER)
    z3 = z2d[:N].reshape(B, NPER)
    dis3 = dis2d[:N].reshape(B, NPER)
    nm, tsc = _topk(saggp3, z3, dis3, bp)

    t2d = jnp.zeros((NP, 1), jnp.float32).at[:N, 0].set(tsc.reshape(N))
    ht16 = _ht(h16, t2d)

    nm_flat = nm.reshape(N)
    selected = nm_flat < NPOOL
    perm = jnp.zeros((NPOOL,), jnp.int32).at[
        jnp.where(selected, nm_flat, NPOOL)].set(
            jnp.arange(N, dtype=jnp.int32), mode="drop")
    xp = ht16[perm, :10]
    em = selected[src] & selected[dst]
    ns = jnp.where(em, nm_flat[src], 0)
    nd = jnp.where(em, nm_flat[dst], 0)
    xl = (xp @ Wg).reshape(NPOOL, HEADS, CH)
    a_s = (xl * att_src).sum(-1)
    a_d = (xl * att_dst).sum(-1)
    loop = jnp.arange(NPOOL, dtype=jnp.int32)
    s_all = jnp.concatenate([ns, loop])
    d_all = jnp.concatenate([nd, loop])
    m_all = jnp.concatenate([em, jnp.ones(NPOOL, bool)])
    logit = jax.nn.leaky_relu(a_s[s_all] + a_d[d_all], 0.2)
    logit = jnp.where(m_all[:, None], logit, -1e9)
    mx = jax.ops.segment_max(logit, d_all, num_segments=NPOOL)
    ex = jnp.exp(logit - mx[d_all])
    ex = jnp.where(m_all[:, None], ex, 0.0)
    den = jax.ops.segment_sum(ex, d_all, num_segments=NPOOL)
    alpha = ex / (den[d_all] + 1e-16)
    alpha_real = alpha[:E]
    alpha_self = alpha[E:]

    xlf = xp @ Wg
    pad0 = jnp.zeros((EP - E,), jnp.int32)
    ns_p = jnp.concatenate([ns, pad0])
    nd_p2 = jnp.concatenate([nd, pad0])
    awT = jnp.zeros((HEADS, EP), jnp.float32).at[:, :E].set(
        alpha_real.T).reshape(HEADS * EP)
    xlT = jnp.zeros((64, NPP), jnp.float32).at[:60, :NPOOL].set(
        xlf.T).reshape(64 * NPP)[:60 * NPP]
    ogp = _gat_msg(ns_p, nd_p2, awT, xlT)
    ogp3 = ogp.reshape(60, NW, NPP)
    xlTp = jnp.zeros((60, NPP), jnp.float32).at[:, :NPOOL].set(xlf.T)
    w3Tp = jnp.zeros((HEADS, NPP), jnp.float32).at[:, :NPOOL].set(alpha_self.T)
    ogt = _combine(ogp3, xlTp, w3Tp, bg.reshape(60, 1))
    ogt3 = ogt[:, :NPOOL].reshape(60, B, K)
    Wn1T = jnp.zeros((32, 60), jnp.float32).at[:30].set(Wn1.T)
    bn1c = jnp.zeros((32, 1), jnp.float32).at[:30, 0].set(bn1)
    Wn2T = jnp.zeros((8, 32), jnp.float32).at[:3, :30].set(Wn2.T)
    bn2c = jnp.zeros((8, 1), jnp.float32).at[:3, 0].set(bn2)
    outT = _headout(ogt3, Wn1T, bn1c, Wn2T, bn2c)
    return outT.T[:, :3]
